# baseline (speedup 1.0000x reference)
import functools

import jax
import jax.numpy as jnp
from jax import lax
from jax.experimental import pallas as pl
from jax.experimental.pallas import tpu as pltpu
from jax.experimental.pallas import tpu_sc as plsc

B = 1
S = 2048
T = B * S
D = 1024
E = 8
F = 2048
V = 50000

_NC = 2
_NS = 16
_NW = _NC * _NS
_ROWS_PER_W = T // _NW


def _sc_gather_rows(table, idx):
    mesh = plsc.VectorSubcoreMesh(core_axis_name="c", subcore_axis_name="s")

    @functools.partial(
        pl.kernel,
        mesh=mesh,
        out_type=jax.ShapeDtypeStruct((T, D), jnp.float32),
        scratch_types=[
            pltpu.VMEM((_ROWS_PER_W,), jnp.int32),
            pltpu.VMEM((_ROWS_PER_W, D), jnp.float32),
            pltpu.SemaphoreType.DMA,
        ],
    )
    def gather_k(table_hbm, idx_hbm, out_hbm, idx_v, rows_v, sem):
        wid = lax.axis_index("s") * _NC + lax.axis_index("c")
        base = wid * _ROWS_PER_W
        pltpu.sync_copy(idx_hbm.at[pl.ds(base, _ROWS_PER_W)], idx_v)
        pltpu.async_copy(table_hbm.at[idx_v], rows_v, sem).wait()
        pltpu.sync_copy(rows_v, out_hbm.at[pl.ds(base, _ROWS_PER_W)])

    return gather_k(table, idx)


def _router_body(x_ref, rw_ref, comb_ref):
    x = x_ref[...].astype(jnp.bfloat16)
    rw = rw_ref[...].astype(jnp.bfloat16)
    logits = lax.dot_general(
        x, rw, (((1,), (0,)), ((), ())),
        preferred_element_type=jnp.float32,
    )
    col = lax.broadcasted_iota(jnp.int32, (T, E), 1)
    m1 = jnp.max(logits, axis=1, keepdims=True)
    i1 = jnp.min(jnp.where(logits == m1, col, E), axis=1, keepdims=True)
    masked = jnp.where(col == i1, -jnp.inf, logits)
    m2 = jnp.max(masked, axis=1, keepdims=True)
    i2 = jnp.min(jnp.where(masked == m2, col, E), axis=1, keepdims=True)
    r = jnp.exp(m2 - m1)
    w_hi = 1.0 / (1.0 + r)
    w_lo = r / (1.0 + r)
    comb_ref[...] = jnp.where(col == i1, w_hi, 0.0) + jnp.where(col == i2, w_lo, 0.0)


def _router(x, router_w):
    return pl.pallas_call(
        _router_body,
        out_shape=jax.ShapeDtypeStruct((T, E), jnp.float32),
    )(x, router_w)


_FBLK = 512
_NF = F // _FBLK


def _moe_body(x_ref, comb_ref, w1_ref, w2_ref, y_ref):
    e = pl.program_id(0)
    f = pl.program_id(1)
    x = x_ref[...]
    w1 = w1_ref[0].astype(jnp.bfloat16)
    w2 = w2_ref[0].astype(jnp.bfloat16)
    h = lax.dot_general(
        x, w1, (((1,), (0,)), ((), ())), preferred_element_type=jnp.float32
    )
    h = jnp.maximum(h, 0.0).astype(jnp.bfloat16)
    part = lax.dot_general(
        h, w2, (((1,), (0,)), ((), ())), preferred_element_type=jnp.float32
    )
    onehot = (lax.broadcasted_iota(jnp.int32, (E, 1), 0) == e).astype(jnp.float32)
    c_col = lax.dot_general(
        comb_ref[...], onehot, (((1,), (0,)), ((), ())),
        preferred_element_type=jnp.float32,
    )
    contrib = part * c_col

    @pl.when(jnp.logical_and(e == 0, f == 0))
    def _init():
        y_ref[...] = contrib

    @pl.when(jnp.logical_or(e > 0, f > 0))
    def _acc():
        y_ref[...] += contrib


def _moe(x_bf, comb, w1, w2):
    return pl.pallas_call(
        _moe_body,
        grid=(E, _NF),
        in_specs=[
            pl.BlockSpec((T, D), lambda e, f: (0, 0)),
            pl.BlockSpec((T, E), lambda e, f: (0, 0)),
            pl.BlockSpec((1, D, _FBLK), lambda e, f: (e, 0, f)),
            pl.BlockSpec((1, _FBLK, D), lambda e, f: (e, f, 0)),
        ],
        out_specs=pl.BlockSpec((T, D), lambda e, f: (0, 0)),
        out_shape=jax.ShapeDtypeStruct((T, D), jnp.float32),
    )(x_bf, comb, w1, w2)


_VBLK = 1024
_NV = (V + _VBLK - 1) // _VBLK


def _proj_body(y_ref, ow_ref, ob_ref, lab_ref, logits_ref, loss_ref,
               m_ref, s_ref, p_ref):
    v = pl.program_id(0)
    y = y_ref[...]
    ow = ow_ref[...].astype(jnp.bfloat16)
    blk = lax.dot_general(
        y, ow, (((1,), (0,)), ((), ())), preferred_element_type=jnp.float32
    ) + ob_ref[...]
    logits_ref[...] = blk

    col = v * _VBLK + lax.broadcasted_iota(jnp.int32, (T, _VBLK), 1)
    valid = col < V
    lm = jnp.where(valid, blk, -jnp.inf)
    bmax = jnp.max(lm, axis=1, keepdims=True)
    bpick = jnp.sum(jnp.where(col == lab_ref[...], blk, 0.0), axis=1, keepdims=True)

    @pl.when(v == 0)
    def _init():
        m_ref[...] = bmax
        s_ref[...] = jnp.sum(jnp.exp(lm - bmax), axis=1, keepdims=True)
        p_ref[...] = bpick

    @pl.when(v > 0)
    def _acc():
        m_old = m_ref[...]
        m_new = jnp.maximum(m_old, bmax)
        s_ref[...] = (s_ref[...] * jnp.exp(m_old - m_new)
                      + jnp.sum(jnp.exp(lm - m_new), axis=1, keepdims=True))
        m_ref[...] = m_new
        p_ref[...] += bpick

    @pl.when(v == _NV - 1)
    def _fin():
        lse = m_ref[...] + jnp.log(s_ref[...])
        loss_ref[0, 0] = jnp.sum(lse - p_ref[...]) * (1.0 / T)


def _proj_loss(y_bf, out_w, out_b2, labels_col):
    return pl.pallas_call(
        _proj_body,
        grid=(_NV,),
        in_specs=[
            pl.BlockSpec((T, D), lambda v: (0, 0)),
            pl.BlockSpec((D, _VBLK), lambda v: (0, v)),
            pl.BlockSpec((1, _VBLK), lambda v: (0, v)),
            pl.BlockSpec((T, 1), lambda v: (0, 0)),
        ],
        out_specs=[
            pl.BlockSpec((T, _VBLK), lambda v: (0, v)),
            pl.BlockSpec(memory_space=pltpu.SMEM),
        ],
        out_shape=[
            jax.ShapeDtypeStruct((T, V), jnp.float32),
            jax.ShapeDtypeStruct((1, 1), jnp.float32),
        ],
        scratch_shapes=[
            pltpu.VMEM((T, 1), jnp.float32),
            pltpu.VMEM((T, 1), jnp.float32),
            pltpu.VMEM((T, 1), jnp.float32),
        ],
    )(y_bf, out_w, out_b2, labels_col)


def kernel(input_ids, labels, emb_table, router_w, w1, w2, out_w, out_b):
    ids_flat = input_ids.reshape(-1).astype(jnp.int32)
    labels_col = labels.reshape(-1, 1).astype(jnp.int32)

    x = _sc_gather_rows(emb_table, ids_flat)
    comb = _router(x, router_w)
    y = _moe(x.astype(jnp.bfloat16), comb, w1, w2)
    logits_flat, loss11 = _proj_loss(
        y.astype(jnp.bfloat16), out_w, out_b.reshape(1, V), labels_col)
    return logits_flat.reshape(B, S, V), loss11.reshape(())


# speedup vs baseline: 1.7521x; 1.7521x over previous
import functools

import jax
import jax.numpy as jnp
from jax import lax
from jax.experimental import pallas as pl
from jax.experimental.pallas import tpu as pltpu
from jax.experimental.pallas import tpu_sc as plsc

B = 1
S = 2048
T = B * S
D = 1024
E = 8
F = 2048
V = 50000

_NC = 2
_NS = 16
_NW = _NC * _NS
_ROWS_PER_W = T // _NW


def _sc_gather_rows(table, idx):
    mesh = plsc.VectorSubcoreMesh(core_axis_name="c", subcore_axis_name="s")

    @functools.partial(
        pl.kernel,
        mesh=mesh,
        out_type=jax.ShapeDtypeStruct((T, D), jnp.float32),
        scratch_types=[
            pltpu.VMEM((_ROWS_PER_W,), jnp.int32),
            pltpu.VMEM((_ROWS_PER_W, D), jnp.float32),
            pltpu.SemaphoreType.DMA,
        ],
    )
    def gather_k(table_hbm, idx_hbm, out_hbm, idx_v, rows_v, sem):
        wid = lax.axis_index("s") * _NC + lax.axis_index("c")
        base = wid * _ROWS_PER_W
        pltpu.sync_copy(idx_hbm.at[pl.ds(base, _ROWS_PER_W)], idx_v)
        pltpu.async_copy(table_hbm.at[idx_v], rows_v, sem).wait()
        pltpu.sync_copy(rows_v, out_hbm.at[pl.ds(base, _ROWS_PER_W)])

    return gather_k(table, idx)


def _router_body(x_ref, rw_ref, comb_ref):
    x = x_ref[...].astype(jnp.bfloat16)
    rw = rw_ref[...].astype(jnp.bfloat16)
    logits = lax.dot_general(
        x, rw, (((1,), (0,)), ((), ())),
        preferred_element_type=jnp.float32,
    )
    col = lax.broadcasted_iota(jnp.int32, (T, E), 1)
    m1 = jnp.max(logits, axis=1, keepdims=True)
    i1 = jnp.min(jnp.where(logits == m1, col, E), axis=1, keepdims=True)
    masked = jnp.where(col == i1, -jnp.inf, logits)
    m2 = jnp.max(masked, axis=1, keepdims=True)
    i2 = jnp.min(jnp.where(masked == m2, col, E), axis=1, keepdims=True)
    r = jnp.exp(m2 - m1)
    w_hi = 1.0 / (1.0 + r)
    w_lo = r / (1.0 + r)
    comb_ref[...] = jnp.where(col == i1, w_hi, 0.0) + jnp.where(col == i2, w_lo, 0.0)


def _router(x, router_w):
    return pl.pallas_call(
        _router_body,
        out_shape=jax.ShapeDtypeStruct((T, E), jnp.float32),
    )(x, router_w)


_FBLK = 512
_NF = F // _FBLK


def _moe_body(x_ref, comb_ref, w1_ref, w2_ref, y_ref):
    e = pl.program_id(0)
    f = pl.program_id(1)
    x = x_ref[...]
    w1 = w1_ref[0].astype(jnp.bfloat16)
    w2 = w2_ref[0].astype(jnp.bfloat16)
    h = lax.dot_general(
        x, w1, (((1,), (0,)), ((), ())), preferred_element_type=jnp.float32
    )
    h = jnp.maximum(h, 0.0).astype(jnp.bfloat16)
    part = lax.dot_general(
        h, w2, (((1,), (0,)), ((), ())), preferred_element_type=jnp.float32
    )
    onehot = (lax.broadcasted_iota(jnp.int32, (E, 1), 0) == e).astype(jnp.float32)
    c_col = lax.dot_general(
        comb_ref[...], onehot, (((1,), (0,)), ((), ())),
        preferred_element_type=jnp.float32,
    )
    contrib = part * c_col

    @pl.when(jnp.logical_and(e == 0, f == 0))
    def _init():
        y_ref[...] = contrib

    @pl.when(jnp.logical_or(e > 0, f > 0))
    def _acc():
        y_ref[...] += contrib


def _moe(x_bf, comb, w1, w2):
    return pl.pallas_call(
        _moe_body,
        grid=(E, _NF),
        in_specs=[
            pl.BlockSpec((T, D), lambda e, f: (0, 0)),
            pl.BlockSpec((T, E), lambda e, f: (0, 0)),
            pl.BlockSpec((1, D, _FBLK), lambda e, f: (e, 0, f)),
            pl.BlockSpec((1, _FBLK, D), lambda e, f: (e, f, 0)),
        ],
        out_specs=pl.BlockSpec((T, D), lambda e, f: (0, 0)),
        out_shape=jax.ShapeDtypeStruct((T, D), jnp.float32),
    )(x_bf, comb, w1, w2)


_VBLK = 1024
_NV = (V + _VBLK - 1) // _VBLK


def _proj_body(yt_ref, owt_ref, ob_ref, lab_ref, logits_ref, loss_ref,
               m_ref, s_ref, p_ref):
    v = pl.program_id(0)
    yt = yt_ref[...]
    owt = owt_ref[...].astype(jnp.bfloat16)
    blk = lax.dot_general(
        owt, yt, (((1,), (0,)), ((), ())), preferred_element_type=jnp.float32
    ) + ob_ref[...]
    logits_ref[...] = blk

    row = v * _VBLK + lax.broadcasted_iota(jnp.int32, (_VBLK, T), 0)
    valid = row < V
    lm = jnp.where(valid, blk, -jnp.inf)
    bmax = jnp.max(lm, axis=0, keepdims=True)
    bpick = jnp.sum(jnp.where(row == lab_ref[...], blk, 0.0), axis=0, keepdims=True)

    @pl.when(v == 0)
    def _init():
        m_ref[...] = bmax
        s_ref[...] = jnp.sum(jnp.exp(lm - bmax), axis=0, keepdims=True)
        p_ref[...] = bpick

    @pl.when(v > 0)
    def _acc():
        m_old = m_ref[...]
        m_new = jnp.maximum(m_old, bmax)
        s_ref[...] = (s_ref[...] * jnp.exp(m_old - m_new)
                      + jnp.sum(jnp.exp(lm - m_new), axis=0, keepdims=True))
        m_ref[...] = m_new
        p_ref[...] += bpick

    @pl.when(v == _NV - 1)
    def _fin():
        lse = m_ref[...] + jnp.log(s_ref[...])
        loss_ref[0, 0] = jnp.sum(lse - p_ref[...]) * (1.0 / T)


def _proj_loss(yt_bf, out_wt, out_b_col, labels_row):
    return pl.pallas_call(
        _proj_body,
        grid=(_NV,),
        in_specs=[
            pl.BlockSpec((D, T), lambda v: (0, 0)),
            pl.BlockSpec((_VBLK, D), lambda v: (v, 0)),
            pl.BlockSpec((_VBLK, 1), lambda v: (v, 0)),
            pl.BlockSpec((1, T), lambda v: (0, 0)),
        ],
        out_specs=[
            pl.BlockSpec((_VBLK, T), lambda v: (v, 0)),
            pl.BlockSpec(memory_space=pltpu.SMEM),
        ],
        out_shape=[
            jax.ShapeDtypeStruct((V, T), jnp.float32),
            jax.ShapeDtypeStruct((1, 1), jnp.float32),
        ],
        scratch_shapes=[
            pltpu.VMEM((1, T), jnp.float32),
            pltpu.VMEM((1, T), jnp.float32),
            pltpu.VMEM((1, T), jnp.float32),
        ],
    )(yt_bf, out_wt, out_b_col, labels_row)


def kernel(input_ids, labels, emb_table, router_w, w1, w2, out_w, out_b):
    ids_flat = input_ids.reshape(-1).astype(jnp.int32)
    labels_row = labels.reshape(1, -1).astype(jnp.int32)

    x = _sc_gather_rows(emb_table, ids_flat)
    comb = _router(x, router_w)
    y = _moe(x.astype(jnp.bfloat16), comb, w1, w2)
    logits_t, loss11 = _proj_loss(
        y.T.astype(jnp.bfloat16), out_w.T, out_b.reshape(V, 1), labels_row)
    return logits_t.T.reshape(B, S, V), loss11.reshape(())
